# baseline (speedup 1.0000x reference)
import functools

import jax
import jax.numpy as jnp
from jax import lax
from jax.experimental import pallas as pl

NC = 32
C = 256
D = 16
K = 16
ALPHA = 8
KB = 8
HS = 32
HM = 32
HMOD = 32
MOD_IN = K + 3 * D + 1
MOD_OUT = K + KB + 1 + D
CPB = 8


def _body(x_ref, h_ref, conn_ref, gate_ref, prev_ref,
          hebb_ref, decay_ref, prim_ref, nid_ref,
          m1_ref, mb1_ref, m2_ref, mb2_ref,
          sw1_ref, sb1_ref, sw2_ref, sb2_ref,
          mw1_ref, mb1s_ref, mw2_ref, mb2s_ref,
          out_ref, *, bs):
    f32 = jnp.float32
    h_all = h_ref[...]
    x_all = x_ref[...]
    gate = jax.nn.sigmoid(gate_ref[...])
    idx = conn_ref[...]
    ciota = lax.broadcasted_iota(jnp.int32, (ALPHA, C), 1)

    gath_cells = []
    for ci in range(CPB):
        h_c = h_all[:, ci]
        h_inj = jnp.concatenate(
            [h_c[:, :ALPHA, :] + x_all[:, ci], h_c[:, ALPHA:, :]], axis=1)
        m_mix = jnp.zeros((bs, ALPHA, C), f32)
        for k in range(K):
            oh_k = (idx[ci, :, k:k + 1] == ciota).astype(f32)
            m_mix = m_mix + gate[:, ci, :, k:k + 1] * oh_k[None]
        g_list = []
        for b in range(bs):
            g_list.append(jnp.dot(m_mix[b], h_inj[b],
                                  preferred_element_type=f32))
        gath_cells.append(jnp.stack(g_list, axis=0))
    gath = jnp.stack(gath_cells, axis=1)

    h_r = h_all[:, :, C - ALPHA:, :]

    msg_inp = jnp.concatenate([h_r, gath, prev_ref[...]], axis=-1)
    flat = msg_inp.reshape(bs * CPB * ALPHA, 3 * D)
    mh = jnp.tanh(
        lax.dot_general(flat, mw1_ref[...], (((1,), (1,)), ((), ())),
                        preferred_element_type=f32) + mb1s_ref[...])
    msg = (lax.dot_general(mh, mw2_ref[...], (((1,), (1,)), ((), ())),
                           preferred_element_type=f32) + mb2s_ref[...])
    msg = msg.reshape(bs, CPB, ALPHA, D)

    nid = jnp.broadcast_to(nid_ref[...][None], (bs, CPB, ALPHA, D))
    mod_inp = jnp.concatenate(
        [hebb_ref[...], h_r, decay_ref[...], prim_ref[...], nid], axis=-1)
    m1 = m1_ref[...].reshape(CPB, ALPHA, HMOD, MOD_IN)
    mb1 = mb1_ref[...].reshape(CPB, ALPHA, HMOD)
    m2 = m2_ref[...].reshape(CPB, ALPHA, HMOD, MOD_OUT)
    mb2 = mb2_ref[...].reshape(CPB, ALPHA, MOD_OUT)
    out_cells = []
    for ci in range(CPB):
        out_list = []
        for r in range(ALPHA):
            hid = jnp.tanh(
                lax.dot_general(mod_inp[:, ci, r, :], m1[ci, r],
                                (((1,), (1,)), ((), ())),
                                preferred_element_type=f32) + mb1[ci, r])
            out_list.append(
                jnp.dot(hid, m2[ci, r], preferred_element_type=f32)
                + mb2[ci, r])
        out_cells.append(jnp.stack(out_list, axis=1))
    outm = jnp.stack(out_cells, axis=1)

    nd = outm[..., K + KB:K + KB + 1]
    new_prim = outm[..., K + KB + 1:]

    st_inp = jnp.concatenate([h_r, msg, new_prim, nd], axis=-1)
    sflat = st_inp.reshape(bs * CPB * ALPHA, 3 * D + 1)
    sh = jnp.tanh(
        lax.dot_general(sflat, sw1_ref[...], (((1,), (1,)), ((), ())),
                        preferred_element_type=f32) + sb1_ref[...])
    delta = (lax.dot_general(sh, sw2_ref[...], (((1,), (1,)), ((), ())),
                             preferred_element_type=f32) + sb2_ref[...])
    delta = delta.reshape(bs, CPB, ALPHA, D)

    out_ref[...] = h_r * jax.nn.sigmoid(nd) + delta


def kernel(x, h, prev_messages, w_conn, decay_logit, primitives_state,
           hebbian_traces, state_w1, state_b1, state_w2, state_b2,
           msg_w1, msg_b1, msg_w2, msg_b2,
           mod_w1, mod_b1, mod_w2, mod_b2,
           neuron_id, conn_indices, border_indices):
    bs = x.shape[0]
    G = C // ALPHA

    dec4 = decay_logit.reshape(bs, NC, C, 1)
    m1 = mod_w1.reshape(NC, G, ALPHA, HMOD, MOD_IN)
    mb1 = mod_b1.reshape(NC, G, ALPHA, HMOD)
    m2 = mod_w2.reshape(NC, G, ALPHA, HMOD, MOD_OUT)
    mb2 = mod_b2.reshape(NC, G, ALPHA, MOD_OUT)

    grid = (NC // CPB,)
    body = functools.partial(_body, bs=bs)
    out = pl.pallas_call(
        body,
        grid=grid,
        in_specs=[
            pl.BlockSpec((bs, CPB, ALPHA, D), lambda i: (0, i, 0, 0)),
            pl.BlockSpec((bs, CPB, C, D), lambda i: (0, i, 0, 0)),
            pl.BlockSpec((CPB, ALPHA, K), lambda i: (i, G - 1, 0)),
            pl.BlockSpec((bs, CPB, ALPHA, K),
                         lambda i: (0, i, G - 1, 0)),
            pl.BlockSpec((bs, CPB, ALPHA, D),
                         lambda i: (0, i, G - 1, 0)),
            pl.BlockSpec((bs, CPB, ALPHA, D),
                         lambda i: (0, i, G - 1, 0)),
            pl.BlockSpec((bs, CPB, ALPHA, 1),
                         lambda i: (0, i, G - 1, 0)),
            pl.BlockSpec((bs, CPB, ALPHA, D),
                         lambda i: (0, i, G - 1, 0)),
            pl.BlockSpec((CPB, ALPHA, D), lambda i: (i, G - 1, 0)),
            pl.BlockSpec((CPB, 1, ALPHA, HMOD, MOD_IN),
                         lambda i: (i, G - 1, 0, 0, 0)),
            pl.BlockSpec((CPB, 1, ALPHA, HMOD), lambda i: (i, G - 1, 0, 0)),
            pl.BlockSpec((CPB, 1, ALPHA, HMOD, MOD_OUT),
                         lambda i: (i, G - 1, 0, 0, 0)),
            pl.BlockSpec((CPB, 1, ALPHA, MOD_OUT), lambda i: (i, G - 1, 0, 0)),
            pl.BlockSpec(state_w1.shape, lambda i: (0, 0)),
            pl.BlockSpec(state_b1.shape, lambda i: (0,)),
            pl.BlockSpec(state_w2.shape, lambda i: (0, 0)),
            pl.BlockSpec(state_b2.shape, lambda i: (0,)),
            pl.BlockSpec(msg_w1.shape, lambda i: (0, 0)),
            pl.BlockSpec(msg_b1.shape, lambda i: (0,)),
            pl.BlockSpec(msg_w2.shape, lambda i: (0, 0)),
            pl.BlockSpec(msg_b2.shape, lambda i: (0,)),
        ],
        out_specs=pl.BlockSpec((bs, CPB, ALPHA, D), lambda i: (0, i, 0, 0)),
        out_shape=jax.ShapeDtypeStruct((bs, NC, ALPHA, D), jnp.float32),
    )(x, h, conn_indices, w_conn, prev_messages,
      hebbian_traces, dec4, primitives_state, neuron_id,
      m1, mb1, m2, mb2,
      state_w1, state_b1, state_w2, state_b2,
      msg_w1, msg_b1, msg_w2, msg_b2)
    return out


# speedup vs baseline: 1.0702x; 1.0702x over previous
import functools

import jax
import jax.numpy as jnp
from jax import lax
from jax.experimental import pallas as pl

NC = 32
C = 256
D = 16
K = 16
ALPHA = 8
KB = 8
HS = 32
HM = 32
HMOD = 32
MOD_IN = K + 3 * D + 1
MOD_OUT = K + KB + 1 + D
CPB = 8


def _body(x_ref, h_ref, conn_ref, gate_ref, prev_ref,
          hebb_ref, decay_ref, prim_ref, nid_ref,
          m1_ref, mb1_ref, m2_ref, mb2_ref,
          sw1_ref, sb1_ref, sw2_ref, sb2_ref,
          mw1_ref, mb1s_ref, mw2_ref, mb2s_ref,
          out_ref, *, bs):
    f32 = jnp.float32
    h_all = h_ref[...]
    x_all = x_ref[...]
    gate = jax.nn.sigmoid(gate_ref[...])
    idx = conn_ref[...]
    ciota = lax.broadcasted_iota(jnp.int32, (ALPHA, C), 1)

    gath_cells = []
    for ci in range(CPB):
        h_c = h_all[:, ci]
        h_inj = jnp.concatenate(
            [h_c[:, :ALPHA, :] + x_all[:, ci], h_c[:, ALPHA:, :]], axis=1)
        m_mix = jnp.zeros((bs, ALPHA, C), f32)
        for k in range(K):
            oh_k = (idx[ci, :, k:k + 1] == ciota).astype(f32)
            m_mix = m_mix + gate[:, ci, :, k:k + 1] * oh_k[None]
        g_list = []
        for b in range(bs):
            g_list.append(jnp.dot(m_mix[b], h_inj[b],
                                  preferred_element_type=f32))
        gath_cells.append(jnp.stack(g_list, axis=0))
    gath = jnp.stack(gath_cells, axis=1)

    h_r = h_all[:, :, C - ALPHA:, :]

    msg_inp = jnp.concatenate([h_r, gath, prev_ref[...]], axis=-1)
    flat = msg_inp.reshape(bs * CPB * ALPHA, 3 * D)
    mh = jnp.tanh(
        lax.dot_general(flat, mw1_ref[...], (((1,), (1,)), ((), ())),
                        preferred_element_type=f32) + mb1s_ref[...])
    msg = (lax.dot_general(mh, mw2_ref[...], (((1,), (1,)), ((), ())),
                           preferred_element_type=f32) + mb2s_ref[...])
    msg = msg.reshape(bs, CPB, ALPHA, D)

    nid = jnp.broadcast_to(nid_ref[...][None], (bs, CPB, ALPHA, D))
    front = jnp.concatenate([hebb_ref[...], h_r], axis=-1)
    back = jnp.concatenate([prim_ref[...], nid], axis=-1)
    dl = decay_ref[...]
    m1 = m1_ref[...].reshape(CPB, ALPHA, HMOD, MOD_IN)
    mb1 = mb1_ref[...].reshape(CPB, ALPHA, HMOD)
    m2 = m2_ref[...].reshape(CPB, ALPHA, HMOD, MOD_OUT)
    mb2 = mb2_ref[...].reshape(CPB, ALPHA, MOD_OUT)
    out_cells = []
    for ci in range(CPB):
        out_list = []
        for r in range(ALPHA):
            inp_r = jnp.concatenate(
                [front[:, ci, r, :],
                 dl[:, ci, C - ALPHA + r:C - ALPHA + r + 1],
                 back[:, ci, r, :]], axis=-1)
            hid = jnp.tanh(
                lax.dot_general(inp_r, m1[ci, r],
                                (((1,), (1,)), ((), ())),
                                preferred_element_type=f32) + mb1[ci, r])
            out_list.append(
                jnp.dot(hid, m2[ci, r], preferred_element_type=f32)
                + mb2[ci, r])
        out_cells.append(jnp.stack(out_list, axis=1))
    outm = jnp.stack(out_cells, axis=1)

    nd = outm[..., K + KB:K + KB + 1]
    new_prim = outm[..., K + KB + 1:]

    st_inp = jnp.concatenate([h_r, msg, new_prim, nd], axis=-1)
    sflat = st_inp.reshape(bs * CPB * ALPHA, 3 * D + 1)
    sh = jnp.tanh(
        lax.dot_general(sflat, sw1_ref[...], (((1,), (1,)), ((), ())),
                        preferred_element_type=f32) + sb1_ref[...])
    delta = (lax.dot_general(sh, sw2_ref[...], (((1,), (1,)), ((), ())),
                             preferred_element_type=f32) + sb2_ref[...])
    delta = delta.reshape(bs, CPB, ALPHA, D)

    out_ref[...] = h_r * jax.nn.sigmoid(nd) + delta


def kernel(x, h, prev_messages, w_conn, decay_logit, primitives_state,
           hebbian_traces, state_w1, state_b1, state_w2, state_b2,
           msg_w1, msg_b1, msg_w2, msg_b2,
           mod_w1, mod_b1, mod_w2, mod_b2,
           neuron_id, conn_indices, border_indices):
    bs = x.shape[0]
    G = C // ALPHA

    m1 = mod_w1.reshape(NC, G, ALPHA, HMOD, MOD_IN)
    mb1 = mod_b1.reshape(NC, G, ALPHA, HMOD)
    m2 = mod_w2.reshape(NC, G, ALPHA, HMOD, MOD_OUT)
    mb2 = mod_b2.reshape(NC, G, ALPHA, MOD_OUT)

    grid = (NC // CPB,)
    body = functools.partial(_body, bs=bs)
    out = pl.pallas_call(
        body,
        grid=grid,
        in_specs=[
            pl.BlockSpec((bs, CPB, ALPHA, D), lambda i: (0, i, 0, 0)),
            pl.BlockSpec((bs, CPB, C, D), lambda i: (0, i, 0, 0)),
            pl.BlockSpec((CPB, ALPHA, K), lambda i: (i, G - 1, 0)),
            pl.BlockSpec((bs, CPB, ALPHA, K),
                         lambda i: (0, i, G - 1, 0)),
            pl.BlockSpec((bs, CPB, ALPHA, D),
                         lambda i: (0, i, G - 1, 0)),
            pl.BlockSpec((bs, CPB, ALPHA, D),
                         lambda i: (0, i, G - 1, 0)),
            pl.BlockSpec((bs, CPB, C), lambda i: (0, i, 0)),
            pl.BlockSpec((bs, CPB, ALPHA, D),
                         lambda i: (0, i, G - 1, 0)),
            pl.BlockSpec((CPB, ALPHA, D), lambda i: (i, G - 1, 0)),
            pl.BlockSpec((CPB, 1, ALPHA, HMOD, MOD_IN),
                         lambda i: (i, G - 1, 0, 0, 0)),
            pl.BlockSpec((CPB, 1, ALPHA, HMOD), lambda i: (i, G - 1, 0, 0)),
            pl.BlockSpec((CPB, 1, ALPHA, HMOD, MOD_OUT),
                         lambda i: (i, G - 1, 0, 0, 0)),
            pl.BlockSpec((CPB, 1, ALPHA, MOD_OUT), lambda i: (i, G - 1, 0, 0)),
            pl.BlockSpec(state_w1.shape, lambda i: (0, 0)),
            pl.BlockSpec(state_b1.shape, lambda i: (0,)),
            pl.BlockSpec(state_w2.shape, lambda i: (0, 0)),
            pl.BlockSpec(state_b2.shape, lambda i: (0,)),
            pl.BlockSpec(msg_w1.shape, lambda i: (0, 0)),
            pl.BlockSpec(msg_b1.shape, lambda i: (0,)),
            pl.BlockSpec(msg_w2.shape, lambda i: (0, 0)),
            pl.BlockSpec(msg_b2.shape, lambda i: (0,)),
        ],
        out_specs=pl.BlockSpec((bs, CPB, ALPHA, D), lambda i: (0, i, 0, 0)),
        out_shape=jax.ShapeDtypeStruct((bs, NC, ALPHA, D), jnp.float32),
    )(x, h, conn_indices, w_conn, prev_messages,
      hebbian_traces, decay_logit, primitives_state, neuron_id,
      m1, mb1, m2, mb2,
      state_w1, state_b1, state_w2, state_b2,
      msg_w1, msg_b1, msg_w2, msg_b2)
    return out


# speedup vs baseline: 2.4001x; 2.2427x over previous
import functools

import jax
import jax.numpy as jnp
from jax import lax
from jax.experimental import pallas as pl

NC = 32
C = 256
D = 16
K = 16
ALPHA = 8
KB = 8
HS = 32
HM = 32
HMOD = 32
MOD_IN = K + 3 * D + 1
MOD_OUT = K + KB + 1 + D
R0 = C - ALPHA
CH = 128
CL = R0 - CH


def _body(xa_ref, h_ref, conn_ref, gate_ref, prev_ref,
          hebb_ref, decay_ref, prim_ref, nid_ref,
          m1_ref, mb1_ref, m2_ref, mb2_ref,
          sw1_ref, sb1_ref, sw2_ref, sb2_ref,
          mw1_ref, mb1s_ref, mw2_ref, mb2s_ref,
          out_ref, *, bs):
    f32 = jnp.float32
    i = pl.program_id(0)

    ht = h_ref[...].reshape(bs, D, C)
    gate = jax.nn.sigmoid(gate_ref[...].reshape(bs, K, C)[:, :, R0:])
    idx = conn_ref[...].reshape(K, C)[:, R0:]

    xa = xa_ref[...]
    cell_mask = (lax.broadcasted_iota(jnp.int32, (1, 1, 1, NC), 3) == i
                 ).astype(f32)
    x_c = (xa * cell_mask).sum(axis=3)

    ciota = lax.broadcasted_iota(jnp.int32, (C, ALPHA), 0)
    m_mix = jnp.zeros((bs, C, ALPHA), f32)
    for k in range(K):
        oh_k = (idx[k:k + 1, :] == ciota).astype(f32)
        m_mix = m_mix + oh_k[None] * gate[:, k:k + 1, :]
    gath_list = []
    for b in range(bs):
        g = jnp.dot(ht[b], m_mix[b], preferred_element_type=f32)
        g = g + lax.dot_general(x_c[b], m_mix[b][:ALPHA, :],
                                (((0,), (0,)), ((), ())),
                                preferred_element_type=f32)
        gath_list.append(g)

    ht_r = ht[:, :, R0:]
    prev_r = prev_ref[...].reshape(bs, D, C)[:, :, R0:]

    inp_cols = [jnp.concatenate([ht_r[b], gath_list[b], prev_r[b]], axis=0)
                for b in range(bs)]
    minp = jnp.concatenate(inp_cols, axis=1)
    mh = jnp.tanh(jnp.dot(mw1_ref[...], minp, preferred_element_type=f32)
                  + mb1s_ref[...])
    msgt = (jnp.dot(mw2_ref[...], mh, preferred_element_type=f32)
            + mb2s_ref[...])

    hebb_r = hebb_ref[...].reshape(bs, K, C)[:, :, R0:]
    prim_r = prim_ref[...].reshape(bs, D, C)[:, :, R0:]
    dec_r = decay_ref[...].reshape(bs, 1, C)[:, :, R0:]
    nid_r = jnp.broadcast_to(
        nid_ref[...].reshape(D, C)[None, :, R0:], (bs, D, ALPHA))
    inp_mod = jnp.concatenate(
        [hebb_r, ht_r, dec_r, prim_r, nid_r], axis=1)

    w1 = m1_ref[...].reshape(MOD_IN, HMOD, CH)
    w2 = m2_ref[...].reshape(MOD_OUT, HMOD, CH)
    hid = jnp.broadcast_to(mb1_ref[...][None, :, CL:], (bs, HMOD, ALPHA))
    for ii in range(MOD_IN):
        hid = hid + inp_mod[:, ii:ii + 1, :] * w1[ii][None, :, CL:]
    hid = jnp.tanh(hid)
    outm = jnp.broadcast_to(mb2_ref[...][None, :, CL:], (bs, MOD_OUT, ALPHA))
    for hh in range(HMOD):
        outm = outm + hid[:, hh:hh + 1, :] * w2[:, hh, CL:][None]

    nd = outm[:, K + KB:K + KB + 1, :]
    new_prim = outm[:, K + KB + 1:, :]

    st_cols = [jnp.concatenate(
        [ht_r[b], msgt[:, b * ALPHA:(b + 1) * ALPHA], new_prim[b], nd[b]],
        axis=0) for b in range(bs)]
    sinp = jnp.concatenate(st_cols, axis=1)
    sh = jnp.tanh(jnp.dot(sw1_ref[...], sinp, preferred_element_type=f32)
                  + sb1_ref[...])
    delta = (jnp.dot(sw2_ref[...], sh, preferred_element_type=f32)
             + sb2_ref[...])

    sig = jax.nn.sigmoid(nd)
    rows = []
    for b in range(bs):
        rows.append(ht_r[b] * sig[b] + delta[:, b * ALPHA:(b + 1) * ALPHA])
    out_ref[...] = jnp.stack(rows, axis=0).reshape(bs, 1, D, ALPHA)


def kernel(x, h, prev_messages, w_conn, decay_logit, primitives_state,
           hebbian_traces, state_w1, state_b1, state_w2, state_b2,
           msg_w1, msg_b1, msg_w2, msg_b2,
           mod_w1, mod_b1, mod_w2, mod_b2,
           neuron_id, conn_indices, border_indices):
    bs = x.shape[0]
    N = NC * C

    xv = x.transpose(0, 2, 3, 1)
    ht = h.transpose(0, 1, 3, 2)
    prevt = prev_messages.transpose(0, 1, 3, 2)
    wct = w_conn.transpose(0, 1, 3, 2)
    hebbt = hebbian_traces.transpose(0, 1, 3, 2)
    primt = primitives_state.transpose(0, 1, 3, 2)
    nidt = neuron_id.transpose(0, 2, 1)
    connt = conn_indices.transpose(0, 2, 1)
    m1t = mod_w1.transpose(2, 1, 0).reshape(MOD_IN * HMOD, N)
    m2t = mod_w2.transpose(2, 1, 0).reshape(MOD_OUT * HMOD, N)
    mb1t = mod_b1.transpose(1, 0)
    mb2t = mod_b2.transpose(1, 0)
    sb1c = state_b1.reshape(HS, 1)
    sb2c = state_b2.reshape(D, 1)
    mb1c = msg_b1.reshape(HM, 1)
    mb2c = msg_b2.reshape(D, 1)

    grid = (NC,)
    body = functools.partial(_body, bs=bs)
    out = pl.pallas_call(
        body,
        grid=grid,
        in_specs=[
            pl.BlockSpec((bs, ALPHA, D, NC), lambda i: (0, 0, 0, 0)),
            pl.BlockSpec((bs, 1, D, C), lambda i: (0, i, 0, 0)),
            pl.BlockSpec((1, K, C), lambda i: (i, 0, 0)),
            pl.BlockSpec((bs, 1, K, C), lambda i: (0, i, 0, 0)),
            pl.BlockSpec((bs, 1, D, C), lambda i: (0, i, 0, 0)),
            pl.BlockSpec((bs, 1, K, C), lambda i: (0, i, 0, 0)),
            pl.BlockSpec((bs, 1, 1, C), lambda i: (0, i, 0, 0)),
            pl.BlockSpec((bs, 1, D, C), lambda i: (0, i, 0, 0)),
            pl.BlockSpec((1, D, C), lambda i: (i, 0, 0)),
            pl.BlockSpec((MOD_IN * HMOD, CH), lambda i: (0, 2 * i + 1)),
            pl.BlockSpec((HMOD, CH), lambda i: (0, 2 * i + 1)),
            pl.BlockSpec((MOD_OUT * HMOD, CH), lambda i: (0, 2 * i + 1)),
            pl.BlockSpec((MOD_OUT, CH), lambda i: (0, 2 * i + 1)),
            pl.BlockSpec(state_w1.shape, lambda i: (0, 0)),
            pl.BlockSpec((HS, 1), lambda i: (0, 0)),
            pl.BlockSpec(state_w2.shape, lambda i: (0, 0)),
            pl.BlockSpec((D, 1), lambda i: (0, 0)),
            pl.BlockSpec(msg_w1.shape, lambda i: (0, 0)),
            pl.BlockSpec((HM, 1), lambda i: (0, 0)),
            pl.BlockSpec(msg_w2.shape, lambda i: (0, 0)),
            pl.BlockSpec((D, 1), lambda i: (0, 0)),
        ],
        out_specs=pl.BlockSpec((bs, 1, D, ALPHA), lambda i: (0, i, 0, 0)),
        out_shape=jax.ShapeDtypeStruct((bs, NC, D, ALPHA), jnp.float32),
    )(xv, ht, connt, wct, prevt, hebbt,
      decay_logit.reshape(bs, NC, 1, C), primt, nidt,
      m1t, mb1t, m2t, mb2t,
      state_w1, sb1c, state_w2, sb2c,
      msg_w1, mb1c, msg_w2, mb2c)
    return out.transpose(0, 1, 3, 2)
